# baseline (speedup 1.0000x reference)
import functools

import jax
import jax.numpy as jnp
from jax import lax
from jax.experimental import pallas as pl
from jax.experimental.pallas import tpu as pltpu
from jax.experimental.pallas import tpu_sc as plsc

NC = 2
NS = 16
NW = NC * NS
CH = 128
CHA = 48


def _fill_1d(ref, n, value):
  v = jnp.full((16,), value, dtype=jnp.float32)

  def body(i, _):
    ref[pl.ds(i * 16, 16)] = v
    return 0

  lax.fori_loop(0, n // 16, body, 0)


def _fill_2d(ref, rows, cols, value):
  v = jnp.full((16,), value, dtype=jnp.float32)

  def body(i, _):
    for j in range(cols // 16):
      ref[i, pl.ds(j * 16, 16)] = v
    return 0

  lax.fori_loop(0, rows, body, 0)


def _deg_body(nk, rpt, dst_hbm, deg_hbm, deg_acc, didx_v, ones_v, zeros_v,
              sem):
  cid = lax.axis_index("c")
  sid = lax.axis_index("s")
  wid = cid * NS + sid

  _fill_1d(ones_v, CH, 1.0)
  _fill_1d(zeros_v, rpt, 0.0)
  pltpu.sync_copy(zeros_v, deg_acc.at[pl.ds(sid * rpt, rpt)])
  pltpu.sync_copy(dst_hbm.at[wid], didx_v)
  plsc.subcore_barrier()

  def fire(k, _):
    pltpu.async_copy(ones_v, deg_acc.at[didx_v.at[k]], sem, add=True)
    return 0

  lax.fori_loop(0, nk, fire, 0)

  def drain(k, _):
    pltpu.make_async_copy(ones_v, deg_acc.at[didx_v.at[0]], sem).wait()
    return 0

  lax.fori_loop(0, nk, drain, 0)
  plsc.subcore_barrier()
  pltpu.sync_copy(deg_acc.at[pl.ds(sid * rpt, rpt)],
                  deg_hbm.at[cid, pl.ds(sid * rpt, rpt)])


NB = 6
NI = 12
GAP = 3


def _copy_rows(src0, dst, base, count):
  for r in range(count // CHA):
    pltpu.sync_copy(src0, dst.at[pl.ds(base + r * CHA, CHA)])
  if count % CHA:
    pltpu.sync_copy(src0.at[pl.ds(0, count % CHA)],
                    dst.at[pl.ds(base + (count // CHA) * CHA, count % CHA)])


def _copy_out(acc, dst, base, count):
  for r in range(count // CHA):
    pltpu.sync_copy(acc.at[pl.ds(base + r * CHA, CHA)],
                    dst.at[pl.ds(base + r * CHA, CHA)])
  if count % CHA:
    rr = (count // CHA) * CHA
    pltpu.sync_copy(acc.at[pl.ds(base + rr, count % CHA)],
                    dst.at[pl.ds(base + rr, count % CHA)])


def _agg_body(nk, rpt, rlast, sd_hbm, hs_hbm, part_hbm,
              acc, idxb_v, rows_v, isem, gsem, ssem):
  cid = lax.axis_index("c")
  sid = lax.axis_index("s")
  wid = cid * NS + sid

  _fill_2d(rows_v.at[0], CHA, 128, 0.0)
  base = sid * rpt

  @pl.when(sid < NS - 1)
  def _():
    _copy_rows(rows_v.at[0], acc, base, rpt)

  @pl.when(sid == NS - 1)
  def _():
    _copy_rows(rows_v.at[0], acc, base, rlast)

  plsc.subcore_barrier()

  for k in range(NI - GAP):
    pltpu.async_copy(sd_hbm.at[wid, k], idxb_v.at[k], isem.at[k])
  for k in range(GAP):
    pltpu.make_async_copy(
        sd_hbm.at[wid, k], idxb_v.at[k], isem.at[k]).wait()
    pltpu.async_copy(hs_hbm.at[idxb_v.at[k, 0]], rows_v.at[k], gsem.at[k])

  def step(k, _):
    b = lax.rem(k, NB)
    i = lax.rem(k, NI)
    pltpu.make_async_copy(
        hs_hbm.at[idxb_v.at[i, 0]], rows_v.at[b], gsem.at[b]).wait()
    pltpu.async_copy(rows_v.at[b], acc.at[idxb_v.at[i, 1]], ssem.at[b],
                     add=True)

    @pl.when(k >= GAP)
    def _():
      bo = lax.rem(k + GAP, NB)
      pltpu.make_async_copy(
          rows_v.at[bo], acc.at[idxb_v.at[0, 1]], ssem.at[bo]).wait()

    @pl.when(k + NI - GAP < nk)
    def _():
      ki = k + NI - GAP
      ii = lax.rem(ki, NI)
      pltpu.async_copy(sd_hbm.at[wid, ki], idxb_v.at[ii], isem.at[ii])

    @pl.when(k + GAP < nk)
    def _():
      kg = k + GAP
      ig = lax.rem(kg, NI)
      bg = lax.rem(kg, NB)
      pltpu.make_async_copy(
          sd_hbm.at[wid, ig], idxb_v.at[ig], isem.at[ig]).wait()
      pltpu.async_copy(hs_hbm.at[idxb_v.at[ig, 0]], rows_v.at[bg],
                       gsem.at[bg])

    return 0

  lax.fori_loop(0, nk, step, 0)
  for t in range(GAP):
    bo = (nk - GAP + t) % NB
    pltpu.make_async_copy(
        rows_v.at[bo], acc.at[idxb_v.at[0, 1]], ssem.at[bo]).wait()
  plsc.subcore_barrier()

  @pl.when(sid < NS - 1)
  def _():
    _copy_out(acc, part_hbm.at[cid], base, rpt)

  @pl.when(sid == NS - 1)
  def _():
    _copy_out(acc, part_hbm.at[cid], base, rlast)


def _mm_body(x_ref, w_ref, h_ref):
  h_ref[...] = jnp.dot(x_ref[...], w_ref[...],
                       preferred_element_type=jnp.float32)


def _hs_body(h_ref, degp_ref, hs_ref):
  deg = degp_ref[:, 0] + degp_ref[:, 1] + 1.0
  dinv = lax.rsqrt(deg)
  hs_ref[...] = h_ref[...] * dinv[:, None]


def _out_body(part_ref, hs_ref, degp_ref, b_ref, out_ref):
  deg = degp_ref[:, 0] + degp_ref[:, 1] + 1.0
  dinv = lax.rsqrt(deg)
  s = part_ref[0] + part_ref[1] + hs_ref[...]
  out_ref[...] = s * dinv[:, None] + b_ref[0, :]


def kernel(x, edge_index, W, b):
  n, d = x.shape
  e = edge_index.shape[1]

  rpt = -(-(n + 1) // NS)
  rpt = -(-rpt // CH) * CH
  np_ = NS * rpt
  rpa = 8 * (-(-(n + 1) // (8 * NS)))
  npa = 8 * (-(-(n + 1) // 8))
  rlast = npa - (NS - 1) * rpa

  nkd = -(-e // (NW * CH))
  epd = NW * nkd * CH
  dstd = jnp.concatenate(
      [edge_index[1], jnp.full((epd - e,), n, dtype=jnp.int32)]
  ).reshape(NW, nkd, CH)

  nk = -(-e // (NW * CHA))
  ep = NW * nk * CHA
  src = jnp.concatenate(
      [edge_index[0], jnp.full((ep - e,), n, dtype=jnp.int32)]
  ).reshape(NW, nk, CHA)
  dst = jnp.concatenate(
      [edge_index[1], jnp.full((ep - e,), n, dtype=jnp.int32)]
  ).reshape(NW, nk, CHA)
  sd = jnp.stack([src, dst], axis=2)
  xp = jnp.concatenate(
      [x, jnp.zeros((np_ - n, d), dtype=x.dtype)])

  mesh = plsc.VectorSubcoreMesh(core_axis_name="c", subcore_axis_name="s")

  deg_kernel = pl.kernel(
      functools.partial(_deg_body, nkd, rpt),
      out_type=jax.ShapeDtypeStruct((NC, np_), jnp.float32),
      mesh=mesh,
      scratch_types=[
          pltpu.VMEM_SHARED((np_,), jnp.float32),
          pltpu.VMEM((nkd, CH), jnp.int32),
          pltpu.VMEM((CH,), jnp.float32),
          pltpu.VMEM((rpt,), jnp.float32),
          pltpu.SemaphoreType.DMA,
      ],
  )
  blk0 = 2048
  h = pl.pallas_call(
      _mm_body,
      grid=(np_ // blk0,),
      in_specs=[
          pl.BlockSpec((blk0, d), lambda i: (i, 0)),
          pl.BlockSpec((d, d), lambda i: (0, 0)),
      ],
      out_specs=pl.BlockSpec((blk0, d), lambda i: (i, 0)),
      out_shape=jax.ShapeDtypeStruct((np_, d), jnp.float32),
  )(xp, W)

  degp = deg_kernel(dstd).T

  blk1 = 2048
  hs = pl.pallas_call(
      _hs_body,
      grid=(np_ // blk1,),
      in_specs=[
          pl.BlockSpec((blk1, d), lambda i: (i, 0)),
          pl.BlockSpec((blk1, NC), lambda i: (i, 0)),
      ],
      out_specs=pl.BlockSpec((blk1, d), lambda i: (i, 0)),
      out_shape=jax.ShapeDtypeStruct((np_, d), jnp.float32),
  )(h, degp)

  agg_kernel = pl.kernel(
      functools.partial(_agg_body, nk, rpa, rlast),
      out_type=jax.ShapeDtypeStruct((NC, npa, d), jnp.float32),
      mesh=mesh,
      scratch_types=[
          pltpu.VMEM_SHARED((npa, d), jnp.float32),
          pltpu.VMEM((NI, 2, CHA), jnp.int32),
          pltpu.VMEM((NB, CHA, d), jnp.float32),
          pltpu.SemaphoreType.DMA((NI,)),
          pltpu.SemaphoreType.DMA((NB,)),
          pltpu.SemaphoreType.DMA((NB,)),
      ],
  )
  part = agg_kernel(sd, hs)

  blk2 = 5000
  out = pl.pallas_call(
      _out_body,
      grid=(n // blk2,),
      in_specs=[
          pl.BlockSpec((NC, blk2, d), lambda i: (0, i, 0)),
          pl.BlockSpec((blk2, d), lambda i: (i, 0)),
          pl.BlockSpec((blk2, NC), lambda i: (i, 0)),
          pl.BlockSpec((1, d), lambda i: (0, 0)),
      ],
      out_specs=pl.BlockSpec((blk2, d), lambda i: (i, 0)),
      out_shape=jax.ShapeDtypeStruct((n, d), jnp.float32),
  )(part, hs, degp, b.reshape(1, d))

  return out


# speedup vs baseline: 1.2750x; 1.2750x over previous
import functools

import jax
import jax.numpy as jnp
from jax import lax
from jax.experimental import pallas as pl
from jax.experimental.pallas import tpu as pltpu
from jax.experimental.pallas import tpu_sc as plsc

NC = 2
NS = 16
NW = NC * NS
CH = 128
CHA = 48


def _fill_1d(ref, n, value):
  v = jnp.full((16,), value, dtype=jnp.float32)

  def body(i, _):
    ref[pl.ds(i * 16, 16)] = v
    return 0

  lax.fori_loop(0, n // 16, body, 0)


def _fill_2d(ref, rows, cols, value):
  v = jnp.full((16,), value, dtype=jnp.float32)

  def body(i, _):
    for j in range(cols // 16):
      ref[i, pl.ds(j * 16, 16)] = v
    return 0

  lax.fori_loop(0, rows, body, 0)


def _deg_body(nk, rpt, dst_hbm, deg_hbm, deg_acc, didx_v, ones_v, zeros_v,
              sem):
  cid = lax.axis_index("c")
  sid = lax.axis_index("s")
  wid = cid * NS + sid

  _fill_1d(ones_v, CH, 1.0)
  _fill_1d(zeros_v, rpt, 0.0)
  pltpu.sync_copy(zeros_v, deg_acc.at[pl.ds(sid * rpt, rpt)])
  pltpu.sync_copy(dst_hbm.at[wid], didx_v)
  plsc.subcore_barrier()

  def fire(k, _):
    pltpu.async_copy(ones_v, deg_acc.at[didx_v.at[k]], sem, add=True)
    return 0

  lax.fori_loop(0, nk, fire, 0)

  def drain(k, _):
    pltpu.make_async_copy(ones_v, deg_acc.at[didx_v.at[0]], sem).wait()
    return 0

  lax.fori_loop(0, nk, drain, 0)
  plsc.subcore_barrier()
  pltpu.sync_copy(deg_acc.at[pl.ds(sid * rpt, rpt)],
                  deg_hbm.at[cid, pl.ds(sid * rpt, rpt)])


NB = 6
NI = 12
GAP = 3


def _copy_rows(src0, dst, base, count):
  for r in range(count // CHA):
    pltpu.sync_copy(src0, dst.at[pl.ds(base + r * CHA, CHA)])
  if count % CHA:
    pltpu.sync_copy(src0.at[pl.ds(0, count % CHA)],
                    dst.at[pl.ds(base + (count // CHA) * CHA, count % CHA)])


def _copy_out(acc, dst, base, count):
  for r in range(count // CHA):
    pltpu.sync_copy(acc.at[pl.ds(base + r * CHA, CHA)],
                    dst.at[pl.ds(base + r * CHA, CHA)])
  if count % CHA:
    rr = (count // CHA) * CHA
    pltpu.sync_copy(acc.at[pl.ds(base + rr, count % CHA)],
                    dst.at[pl.ds(base + rr, count % CHA)])


def _agg_body(nk0, nk1, rpt, rlast, sd_hbm, hs_hbm, part_hbm,
              acc, idxb_v, rows_v, isem, gsem, ssem):
  cid = lax.axis_index("c")
  sid = lax.axis_index("s")
  nk = jnp.where(cid == 0, nk0, nk1)
  cbase = cid * (NS * nk0) + sid * nk

  _fill_2d(rows_v.at[0], CHA, 128, 0.0)
  base = sid * rpt

  @pl.when(sid < NS - 1)
  def _():
    _copy_rows(rows_v.at[0], acc, base, rpt)

  @pl.when(sid == NS - 1)
  def _():
    _copy_rows(rows_v.at[0], acc, base, rlast)

  plsc.subcore_barrier()

  for k in range(NI - GAP):
    pltpu.async_copy(sd_hbm.at[cbase + k], idxb_v.at[k], isem.at[k])
  for k in range(GAP):
    pltpu.make_async_copy(
        sd_hbm.at[cbase + k], idxb_v.at[k], isem.at[k]).wait()
    pltpu.async_copy(hs_hbm.at[idxb_v.at[k, 0]], rows_v.at[k], gsem.at[k])

  def step(k, _):
    b = lax.rem(k, NB)
    i = lax.rem(k, NI)
    pltpu.make_async_copy(
        hs_hbm.at[idxb_v.at[i, 0]], rows_v.at[b], gsem.at[b]).wait()
    pltpu.async_copy(rows_v.at[b], acc.at[idxb_v.at[i, 1]], ssem.at[b],
                     add=True)

    @pl.when(k >= GAP)
    def _():
      bo = lax.rem(k + GAP, NB)
      pltpu.make_async_copy(
          rows_v.at[bo], acc.at[idxb_v.at[0, 1]], ssem.at[bo]).wait()

    @pl.when(k + NI - GAP < nk)
    def _():
      ki = k + NI - GAP
      ii = lax.rem(ki, NI)
      pltpu.async_copy(sd_hbm.at[cbase + ki], idxb_v.at[ii], isem.at[ii])

    @pl.when(k + GAP < nk)
    def _():
      kg = k + GAP
      ig = lax.rem(kg, NI)
      bg = lax.rem(kg, NB)
      pltpu.make_async_copy(
          sd_hbm.at[cbase], idxb_v.at[ig], isem.at[ig]).wait()
      pltpu.async_copy(hs_hbm.at[idxb_v.at[ig, 0]], rows_v.at[bg],
                       gsem.at[bg])

    return 0

  lax.fori_loop(0, nk, step, 0)
  for t in range(GAP):
    bo = lax.rem(nk - GAP + t, NB)
    pltpu.make_async_copy(
        rows_v.at[bo], acc.at[idxb_v.at[0, 1]], ssem.at[bo]).wait()
  plsc.subcore_barrier()

  @pl.when(sid < NS - 1)
  def _():
    _copy_out(acc, part_hbm.at[cid], base, rpt)

  @pl.when(sid == NS - 1)
  def _():
    _copy_out(acc, part_hbm.at[cid], base, rlast)


def _mm_body(x_ref, w_ref, h_ref):
  h_ref[...] = jnp.dot(x_ref[...], w_ref[...],
                       preferred_element_type=jnp.float32)


def _hs_body(blk, h_ref, degp_ref, hs_ref, dinv_ref):
  sl = pl.ds(pl.multiple_of(pl.program_id(0) * blk, 128), blk)
  deg = degp_ref[0, sl] + degp_ref[1, sl] + 1.0
  dinv = lax.rsqrt(deg)
  hs_ref[...] = h_ref[...] * dinv[:, None]
  dinv_ref[...] = dinv[:, None]


def _out_body(part_ref, hs_ref, dinv_ref, b_ref, out_ref):
  s = part_ref[0] + part_ref[1] + hs_ref[...]
  out_ref[...] = s * dinv_ref[...] + b_ref[0, :]


def kernel(x, edge_index, W, b):
  n, d = x.shape
  e = edge_index.shape[1]

  rpt = -(-(n + 1) // NS)
  rpt = -(-rpt // CH) * CH
  np_ = NS * rpt
  rpa = 8 * (-(-(n + 1) // (8 * NS)))
  npa = 8 * (-(-(n + 1) // 8))
  rlast = npa - (NS - 1) * rpa

  nkd = -(-e // (NW * CH))
  epd = NW * nkd * CH
  dstd = jnp.concatenate(
      [edge_index[1], jnp.full((epd - e,), n, dtype=jnp.int32)]
  ).reshape(NW, nkd, CH)

  F0 = 0.578
  nkt = -(-e // CHA)
  nk0 = -(-int(F0 * nkt) // NS)
  nk1 = max(1, -(-(nkt - NS * nk0) // NS))
  nkt = NS * (nk0 + nk1)
  ep = nkt * CHA
  src = jnp.concatenate(
      [edge_index[0], jnp.full((ep - e,), n, dtype=jnp.int32)]
  ).reshape(nkt, CHA)
  dst = jnp.concatenate(
      [edge_index[1], jnp.full((ep - e,), n, dtype=jnp.int32)]
  ).reshape(nkt, CHA)
  sd = jnp.stack([src, dst], axis=1)
  xp = jnp.concatenate(
      [x, jnp.zeros((np_ - n, d), dtype=x.dtype)])

  mesh = plsc.VectorSubcoreMesh(core_axis_name="c", subcore_axis_name="s")

  deg_kernel = pl.kernel(
      functools.partial(_deg_body, nkd, rpt),
      out_type=jax.ShapeDtypeStruct((NC, np_), jnp.float32),
      mesh=mesh,
      scratch_types=[
          pltpu.VMEM_SHARED((np_,), jnp.float32),
          pltpu.VMEM((nkd, CH), jnp.int32),
          pltpu.VMEM((CH,), jnp.float32),
          pltpu.VMEM((rpt,), jnp.float32),
          pltpu.SemaphoreType.DMA,
      ],
  )
  blk0 = 2048
  h = pl.pallas_call(
      _mm_body,
      grid=(np_ // blk0,),
      in_specs=[
          pl.BlockSpec((blk0, d), lambda i: (i, 0)),
          pl.BlockSpec((d, d), lambda i: (0, 0)),
      ],
      out_specs=pl.BlockSpec((blk0, d), lambda i: (i, 0)),
      out_shape=jax.ShapeDtypeStruct((np_, d), jnp.float32),
  )(xp, W)

  degp = deg_kernel(dstd)

  blk1 = 2048
  hs, dinv = pl.pallas_call(
      functools.partial(_hs_body, blk1),
      grid=(np_ // blk1,),
      in_specs=[
          pl.BlockSpec((blk1, d), lambda i: (i, 0)),
          pl.BlockSpec((NC, np_), lambda i: (0, 0)),
      ],
      out_specs=[
          pl.BlockSpec((blk1, d), lambda i: (i, 0)),
          pl.BlockSpec((blk1, 1), lambda i: (i, 0)),
      ],
      out_shape=[
          jax.ShapeDtypeStruct((np_, d), jnp.float32),
          jax.ShapeDtypeStruct((np_, 1), jnp.float32),
      ],
  )(h, degp)

  agg_kernel = pl.kernel(
      functools.partial(_agg_body, nk0, nk1, rpa, rlast),
      out_type=jax.ShapeDtypeStruct((NC, npa, d), jnp.float32),
      mesh=mesh,
      scratch_types=[
          pltpu.VMEM_SHARED((npa, d), jnp.float32),
          pltpu.VMEM((NI, 2, CHA), jnp.int32),
          pltpu.VMEM((NB, CHA, d), jnp.float32),
          pltpu.SemaphoreType.DMA((NI,)),
          pltpu.SemaphoreType.DMA((NB,)),
          pltpu.SemaphoreType.DMA((NB,)),
      ],
  )
  part = agg_kernel(sd, hs)

  blk2 = 5000
  out = pl.pallas_call(
      _out_body,
      grid=(n // blk2,),
      in_specs=[
          pl.BlockSpec((NC, blk2, d), lambda i: (0, i, 0)),
          pl.BlockSpec((blk2, d), lambda i: (i, 0)),
          pl.BlockSpec((blk2, 1), lambda i: (i, 0)),
          pl.BlockSpec((1, d), lambda i: (0, 0)),
      ],
      out_specs=pl.BlockSpec((blk2, d), lambda i: (i, 0)),
      out_shape=jax.ShapeDtypeStruct((n, d), jnp.float32),
  )(part, hs, dinv, b.reshape(1, d))

  return out


# speedup vs baseline: 1.3190x; 1.0345x over previous
import functools

import jax
import jax.numpy as jnp
from jax import lax
from jax.experimental import pallas as pl
from jax.experimental.pallas import tpu as pltpu
from jax.experimental.pallas import tpu_sc as plsc

NC = 2
NS = 16
NW = NC * NS
CH = 128
CHA = 64
NB = 5
NI = 10
GAP = 2
F0 = 0.62


def _fill_1d(ref, n, value):
  v = jnp.full((16,), value, dtype=jnp.float32)

  def body(i, _):
    ref[pl.ds(i * 16, 16)] = v
    return 0

  lax.fori_loop(0, n // 16, body, 0)


def _fill_2d(ref, rows, cols, value):
  v = jnp.full((16,), value, dtype=jnp.float32)

  def body(i, _):
    for j in range(cols // 16):
      ref[i, pl.ds(j * 16, 16)] = v
    return 0

  lax.fori_loop(0, rows, body, 0)


def _deg_body(b2, r2, rpt, dst_hbm, deg_hbm, deg_acc, didx_v, ones_v,
              zeros_v, isem, sem):
  cid = lax.axis_index("c")
  sid = lax.axis_index("s")
  wid = cid * NS + sid
  cnt = b2 + jnp.where(wid < r2, 1, 0)
  start = wid * b2 + jnp.minimum(wid, r2)

  _fill_1d(ones_v, CH, 1.0)
  _fill_1d(zeros_v, rpt, 0.0)
  pltpu.sync_copy(zeros_v, deg_acc.at[pl.ds(sid * rpt, rpt)])

  def load(k, _):
    pltpu.async_copy(dst_hbm.at[pl.ds((start + k) * CH, CH)], didx_v.at[k],
                     isem)
    return 0

  lax.fori_loop(0, cnt, load, 0)

  def load_drain(k, _):
    pltpu.make_async_copy(dst_hbm.at[pl.ds(0, CH)], didx_v.at[0],
                          isem).wait()
    return 0

  lax.fori_loop(0, cnt, load_drain, 0)
  plsc.subcore_barrier()

  def fire(k, _):
    pltpu.async_copy(ones_v, deg_acc.at[didx_v.at[k]], sem, add=True)
    return 0

  lax.fori_loop(0, cnt, fire, 0)

  def drain(k, _):
    pltpu.make_async_copy(ones_v, deg_acc.at[didx_v.at[0]], sem).wait()
    return 0

  lax.fori_loop(0, cnt, drain, 0)
  plsc.subcore_barrier()
  pltpu.sync_copy(deg_acc.at[pl.ds(sid * rpt, rpt)],
                  deg_hbm.at[cid, pl.ds(sid * rpt, rpt)])


def _copy_rows(src0, dst, base, count):
  for r in range(count // CHA):
    pltpu.sync_copy(src0, dst.at[pl.ds(base + r * CHA, CHA)])
  if count % CHA:
    pltpu.sync_copy(src0.at[pl.ds(0, count % CHA)],
                    dst.at[pl.ds(base + (count // CHA) * CHA, count % CHA)])


def _copy_out(acc, dst, base, count):
  for r in range(count // CHA):
    pltpu.sync_copy(acc.at[pl.ds(base + r * CHA, CHA)],
                    dst.at[pl.ds(base + r * CHA, CHA)])
  if count % CHA:
    rr = (count // CHA) * CHA
    pltpu.sync_copy(acc.at[pl.ds(base + rr, count % CHA)],
                    dst.at[pl.ds(base + rr, count % CHA)])


def _agg_body(k0, b0, r0, b1, r1, rpt, rlast, srcr_hbm, dstr_hbm, hs_hbm,
              part_hbm, acc, sidx_v, didx_v, rows_v, isem, jsem, gsem, ssem):
  cid = lax.axis_index("c")
  sid = lax.axis_index("s")
  cnt = jnp.where(cid == 0,
                  b0 + jnp.where(sid < r0, 1, 0),
                  b1 + jnp.where(sid < r1, 1, 0))
  start = jnp.where(cid == 0,
                    sid * b0 + jnp.minimum(sid, r0),
                    k0 + sid * b1 + jnp.minimum(sid, r1))

  _fill_2d(rows_v.at[0], CHA, 128, 0.0)
  base = sid * rpt

  @pl.when(sid < NS - 1)
  def _():
    _copy_rows(rows_v.at[0], acc, base, rpt)

  @pl.when(sid == NS - 1)
  def _():
    _copy_rows(rows_v.at[0], acc, base, rlast)

  plsc.subcore_barrier()

  def fire_idx(k, slot):
    pltpu.async_copy(srcr_hbm.at[pl.ds((start + k) * CHA, CHA)],
                     sidx_v.at[slot], isem.at[slot])
    pltpu.async_copy(dstr_hbm.at[pl.ds((start + k) * CHA, CHA)],
                     didx_v.at[slot], jsem.at[slot])

  def wait_isem(slot):
    pltpu.make_async_copy(
        srcr_hbm.at[pl.ds(0, CHA)], sidx_v.at[slot], isem.at[slot]).wait()

  def wait_jsem(slot):
    pltpu.make_async_copy(
        dstr_hbm.at[pl.ds(0, CHA)], didx_v.at[slot], jsem.at[slot]).wait()

  def fire_gather(k, slot, b):
    pltpu.async_copy(hs_hbm.at[sidx_v.at[slot]], rows_v.at[b], gsem.at[b])

  def wait_gather(b):
    pltpu.make_async_copy(
        hs_hbm.at[sidx_v.at[0]], rows_v.at[b], gsem.at[b]).wait()

  def wait_scatter(b):
    pltpu.make_async_copy(
        rows_v.at[b], acc.at[didx_v.at[0]], ssem.at[b]).wait()

  for k in range(NI - GAP):
    fire_idx(k, k)
  for k in range(GAP):
    wait_isem(k)
    fire_gather(k, k, k)

  def step(k, _):
    b = lax.rem(k, NB)
    i = lax.rem(k, NI)
    wait_gather(b)
    wait_jsem(i)
    pltpu.async_copy(rows_v.at[b], acc.at[didx_v.at[i]], ssem.at[b],
                     add=True)

    @pl.when(k >= GAP)
    def _():
      wait_scatter(lax.rem(k + NB - GAP, NB))

    @pl.when(k + NI - GAP < cnt)
    def _():
      ki = k + NI - GAP
      fire_idx(ki, lax.rem(ki, NI))

    @pl.when(k + GAP < cnt)
    def _():
      kg = k + GAP
      ig = lax.rem(kg, NI)
      wait_isem(ig)
      fire_gather(kg, ig, lax.rem(kg, NB))

    return 0

  lax.fori_loop(0, cnt, step, 0)
  for t in range(GAP):
    wait_scatter(lax.rem(cnt - GAP + t, NB))
  plsc.subcore_barrier()

  @pl.when(sid < NS - 1)
  def _():
    _copy_out(acc, part_hbm.at[cid], base, rpt)

  @pl.when(sid == NS - 1)
  def _():
    _copy_out(acc, part_hbm.at[cid], base, rlast)


def _hs_body(blk, x_ref, w_ref, degp_ref, hs_ref, dinv_ref):
  sl = pl.ds(pl.multiple_of(pl.program_id(0) * blk, 128), blk)
  deg = degp_ref[0, sl] + degp_ref[1, sl] + 1.0
  dinv = lax.rsqrt(deg)
  h = jnp.dot(x_ref[...], w_ref[...], preferred_element_type=jnp.float32)
  hs_ref[...] = h * dinv[:, None]
  dinv_ref[...] = dinv[:, None]


def _out_body(part_ref, hs_ref, dinv_ref, b_ref, out_ref):
  s = part_ref[0] + part_ref[1] + hs_ref[...]
  out_ref[...] = s * dinv_ref[...] + b_ref[0, :]


def kernel(x, edge_index, W, b):
  n, d = x.shape
  e = edge_index.shape[1]

  rpt = -(-(n + 1) // NS)
  rpt = -(-rpt // CH) * CH
  np_ = NS * rpt
  rpa = 8 * (-(-(n + 1) // (8 * NS)))
  npa = 8 * (-(-(n + 1) // 8))
  rlast = npa - (NS - 1) * rpa

  f2 = -(-e // CH)
  dst2 = edge_index[1]
  if f2 * CH != e:
    dst2 = jnp.concatenate(
        [dst2, jnp.full((f2 * CH - e,), n, dtype=jnp.int32)])
  b2, r2 = f2 // NW, f2 % NW

  f = -(-e // CHA)
  srcr, dstr = edge_index[0], edge_index[1]
  if f * CHA != e:
    padv = jnp.full((f * CHA - e,), n, dtype=jnp.int32)
    srcr = jnp.concatenate([srcr, padv])
    dstr = jnp.concatenate([dstr, padv])
  k0 = int(round(F0 * f))
  k1 = f - k0
  b0, r0 = k0 // NS, k0 % NS
  b1, r1 = k1 // NS, k1 % NS

  xp = jnp.concatenate([x, jnp.zeros((np_ - n, d), dtype=x.dtype)])

  mesh = plsc.VectorSubcoreMesh(core_axis_name="c", subcore_axis_name="s")

  deg_kernel = pl.kernel(
      functools.partial(_deg_body, b2, r2, rpt),
      out_type=jax.ShapeDtypeStruct((NC, np_), jnp.float32),
      mesh=mesh,
      scratch_types=[
          pltpu.VMEM_SHARED((np_,), jnp.float32),
          pltpu.VMEM((b2 + (1 if r2 else 0), CH), jnp.int32),
          pltpu.VMEM((CH,), jnp.float32),
          pltpu.VMEM((rpt,), jnp.float32),
          pltpu.SemaphoreType.DMA,
          pltpu.SemaphoreType.DMA,
      ],
  )
  degp = deg_kernel(dst2)

  blk1 = 2048
  hs, dinv = pl.pallas_call(
      functools.partial(_hs_body, blk1),
      grid=(np_ // blk1,),
      in_specs=[
          pl.BlockSpec((blk1, d), lambda i: (i, 0)),
          pl.BlockSpec((d, d), lambda i: (0, 0)),
          pl.BlockSpec((NC, np_), lambda i: (0, 0)),
      ],
      out_specs=[
          pl.BlockSpec((blk1, d), lambda i: (i, 0)),
          pl.BlockSpec((blk1, 1), lambda i: (i, 0)),
      ],
      out_shape=[
          jax.ShapeDtypeStruct((np_, d), jnp.float32),
          jax.ShapeDtypeStruct((np_, 1), jnp.float32),
      ],
  )(xp, W, degp)

  agg_kernel = pl.kernel(
      functools.partial(_agg_body, k0, b0, r0, b1, r1, rpa, rlast),
      out_type=jax.ShapeDtypeStruct((NC, npa, d), jnp.float32),
      mesh=mesh,
      scratch_types=[
          pltpu.VMEM_SHARED((npa, d), jnp.float32),
          pltpu.VMEM((NI, CHA), jnp.int32),
          pltpu.VMEM((NI, CHA), jnp.int32),
          pltpu.VMEM((NB, CHA, d), jnp.float32),
          pltpu.SemaphoreType.DMA((NI,)),
          pltpu.SemaphoreType.DMA((NI,)),
          pltpu.SemaphoreType.DMA((NB,)),
          pltpu.SemaphoreType.DMA((NB,)),
      ],
  )
  part = agg_kernel(srcr, dstr, hs)

  blk2 = 5000
  out = pl.pallas_call(
      _out_body,
      grid=(n // blk2,),
      in_specs=[
          pl.BlockSpec((NC, blk2, d), lambda i: (0, i, 0)),
          pl.BlockSpec((blk2, d), lambda i: (i, 0)),
          pl.BlockSpec((blk2, 1), lambda i: (i, 0)),
          pl.BlockSpec((1, d), lambda i: (0, 0)),
      ],
      out_specs=pl.BlockSpec((blk2, d), lambda i: (i, 0)),
      out_shape=jax.ShapeDtypeStruct((n, d), jnp.float32),
  )(part, hs, dinv, b.reshape(1, d))

  return out


# speedup vs baseline: 1.4646x; 1.1104x over previous
import functools

import jax
import jax.numpy as jnp
from jax import lax
from jax.experimental import pallas as pl
from jax.experimental.pallas import tpu as pltpu
from jax.experimental.pallas import tpu_sc as plsc

NC = 2
NS = 16
NW = NC * NS
CH = 128
CHA = 64
NB = 5
NI = 10
GAP = 2
F0 = 0.52


def _fill_1d(ref, n, value):
  v = jnp.full((16,), value, dtype=jnp.float32)

  def body(i, _):
    ref[pl.ds(i * 16, 16)] = v
    return 0

  lax.fori_loop(0, n // 16, body, 0)


def _fill_2d(ref, rows, cols, value):
  v = jnp.full((16,), value, dtype=jnp.float32)

  def body(i, _):
    for j in range(cols // 16):
      ref[i, pl.ds(j * 16, 16)] = v
    return 0

  lax.fori_loop(0, rows, body, 0)


def _deg_body(b2, r2, rpt, dst_hbm, deg_hbm, deg_acc, didx_v, ones_v,
              zeros_v, isem, sem):
  cid = lax.axis_index("c")
  sid = lax.axis_index("s")
  wid = cid * NS + sid
  cnt = b2 + jnp.where(wid < r2, 1, 0)
  start = wid * b2 + jnp.minimum(wid, r2)

  _fill_1d(ones_v, CH, 1.0)
  _fill_1d(zeros_v, rpt, 0.0)
  pltpu.sync_copy(zeros_v, deg_acc.at[pl.ds(sid * rpt, rpt)])

  def load(k, _):
    pltpu.async_copy(dst_hbm.at[pl.ds((start + k) * CH, CH)], didx_v.at[k],
                     isem)
    return 0

  lax.fori_loop(0, cnt, load, 0)

  def load_drain(k, _):
    pltpu.make_async_copy(dst_hbm.at[pl.ds(0, CH)], didx_v.at[0],
                          isem).wait()
    return 0

  lax.fori_loop(0, cnt, load_drain, 0)
  plsc.subcore_barrier()

  def fire(k, _):
    pltpu.async_copy(ones_v, deg_acc.at[didx_v.at[k]], sem, add=True)
    return 0

  lax.fori_loop(0, cnt, fire, 0)

  def drain(k, _):
    pltpu.make_async_copy(ones_v, deg_acc.at[didx_v.at[0]], sem).wait()
    return 0

  lax.fori_loop(0, cnt, drain, 0)
  plsc.subcore_barrier()
  pltpu.sync_copy(deg_acc.at[pl.ds(sid * rpt, rpt)],
                  deg_hbm.at[cid, pl.ds(sid * rpt, rpt)])


def _copy_rows(src0, dst, base, count):
  for r in range(count // CHA):
    pltpu.sync_copy(src0, dst.at[pl.ds(base + r * CHA, CHA)])
  if count % CHA:
    pltpu.sync_copy(src0.at[pl.ds(0, count % CHA)],
                    dst.at[pl.ds(base + (count // CHA) * CHA, count % CHA)])


def _copy_out(acc, dst, base, count):
  for r in range(count // CHA):
    pltpu.sync_copy(acc.at[pl.ds(base + r * CHA, CHA)],
                    dst.at[pl.ds(base + r * CHA, CHA)])
  if count % CHA:
    rr = (count // CHA) * CHA
    pltpu.sync_copy(acc.at[pl.ds(base + rr, count % CHA)],
                    dst.at[pl.ds(base + rr, count % CHA)])


def _agg_body(k0, b0, r0, b1, r1, rpt, rlast, srcr_hbm, dstr_hbm, hs_hbm,
              part_hbm, acc, sidx_v, didx_v, rows_v, isem, jsem, gsem, ssem):
  cid = lax.axis_index("c")
  sid = lax.axis_index("s")
  cnt = jnp.where(cid == 0,
                  b0 + jnp.where(sid < r0, 1, 0),
                  b1 + jnp.where(sid < r1, 1, 0))
  start = jnp.where(cid == 0,
                    sid * b0 + jnp.minimum(sid, r0),
                    k0 + sid * b1 + jnp.minimum(sid, r1))

  _fill_2d(rows_v.at[0], CHA, 128, 0.0)
  base = sid * rpt

  @pl.when(sid < NS - 1)
  def _():
    _copy_rows(rows_v.at[0], acc, base, rpt)

  @pl.when(sid == NS - 1)
  def _():
    _copy_rows(rows_v.at[0], acc, base, rlast)

  plsc.subcore_barrier()

  def fire_idx(k, slot):
    pltpu.async_copy(srcr_hbm.at[pl.ds((start + k) * CHA, CHA)],
                     sidx_v.at[slot], isem.at[slot])
    pltpu.async_copy(dstr_hbm.at[pl.ds((start + k) * CHA, CHA)],
                     didx_v.at[slot], jsem.at[slot])

  def wait_isem(slot):
    pltpu.make_async_copy(
        srcr_hbm.at[pl.ds(0, CHA)], sidx_v.at[slot], isem.at[slot]).wait()

  def wait_jsem(slot):
    pltpu.make_async_copy(
        dstr_hbm.at[pl.ds(0, CHA)], didx_v.at[slot], jsem.at[slot]).wait()

  def fire_gather(k, slot, b):
    pltpu.async_copy(hs_hbm.at[sidx_v.at[slot]], rows_v.at[b], gsem.at[b])

  def wait_gather(b):
    pltpu.make_async_copy(
        hs_hbm.at[sidx_v.at[0]], rows_v.at[b], gsem.at[b]).wait()

  def wait_scatter(b):
    pltpu.make_async_copy(
        rows_v.at[b], acc.at[didx_v.at[0]], ssem.at[b]).wait()

  for k in range(NI - GAP):
    fire_idx(k, k)
  for k in range(GAP):
    wait_isem(k)
    fire_gather(k, k, k)

  def step(k, _):
    b = lax.rem(k, NB)
    i = lax.rem(k, NI)
    wait_gather(b)
    wait_jsem(i)
    pltpu.async_copy(rows_v.at[b], acc.at[didx_v.at[i]], ssem.at[b],
                     add=True)

    @pl.when(k >= GAP)
    def _():
      wait_scatter(lax.rem(k + NB - GAP, NB))

    @pl.when(k + NI - GAP < cnt)
    def _():
      ki = k + NI - GAP
      fire_idx(ki, lax.rem(ki, NI))

    @pl.when(k + GAP < cnt)
    def _():
      kg = k + GAP
      ig = lax.rem(kg, NI)
      wait_isem(ig)
      fire_gather(kg, ig, lax.rem(kg, NB))

    return 0

  lax.fori_loop(0, cnt, step, 0)
  for t in range(GAP):
    wait_scatter(lax.rem(cnt - GAP + t, NB))
  plsc.subcore_barrier()

  @pl.when(sid < NS - 1)
  def _():
    _copy_out(acc, part_hbm.at[cid], base, rpt)

  @pl.when(sid == NS - 1)
  def _():
    _copy_out(acc, part_hbm.at[cid], base, rlast)


def _hs_body(blk, x_ref, w_ref, degp_ref, hs_ref, dinv_ref):
  sl = pl.ds(pl.multiple_of(pl.program_id(0) * blk, 128), blk)
  deg = degp_ref[0, sl] + degp_ref[1, sl] + 1.0
  dinv = lax.rsqrt(deg)
  h = jnp.dot(x_ref[...], w_ref[...], preferred_element_type=jnp.float32)
  hs_ref[...] = h * dinv[:, None]
  dinv_ref[...] = dinv[:, None]


def _out_body(part_ref, hs_ref, dinv_ref, b_ref, out_ref):
  s = part_ref[0] + part_ref[1] + hs_ref[...]
  out_ref[...] = s * dinv_ref[...] + b_ref[0, :]


def kernel(x, edge_index, W, b):
  n, d = x.shape
  e = edge_index.shape[1]

  rpt = -(-(n + 1) // NS)
  rpt = -(-rpt // CH) * CH
  np_ = NS * rpt
  rpa = 8 * (-(-(n + 1) // (8 * NS)))
  npa = 8 * (-(-(n + 1) // 8))
  rlast = npa - (NS - 1) * rpa

  f2 = -(-e // CH)
  dst2 = edge_index[1]
  if f2 * CH != e:
    dst2 = jnp.concatenate(
        [dst2, jnp.full((f2 * CH - e,), n, dtype=jnp.int32)])
  b2, r2 = f2 // NW, f2 % NW

  f = -(-e // CHA)
  srcr, dstr = edge_index[0], edge_index[1]
  if f * CHA != e:
    padv = jnp.full((f * CHA - e,), n, dtype=jnp.int32)
    srcr = jnp.concatenate([srcr, padv])
    dstr = jnp.concatenate([dstr, padv])
  k0 = int(round(F0 * f))
  k1 = f - k0
  b0, r0 = k0 // NS, k0 % NS
  b1, r1 = k1 // NS, k1 % NS

  xp = jnp.concatenate([x, jnp.zeros((np_ - n, d), dtype=x.dtype)])

  mesh = plsc.VectorSubcoreMesh(core_axis_name="c", subcore_axis_name="s")

  deg_kernel = pl.kernel(
      functools.partial(_deg_body, b2, r2, rpt),
      out_type=jax.ShapeDtypeStruct((NC, np_), jnp.float32),
      mesh=mesh,
      scratch_types=[
          pltpu.VMEM_SHARED((np_,), jnp.float32),
          pltpu.VMEM((b2 + (1 if r2 else 0), CH), jnp.int32),
          pltpu.VMEM((CH,), jnp.float32),
          pltpu.VMEM((rpt,), jnp.float32),
          pltpu.SemaphoreType.DMA,
          pltpu.SemaphoreType.DMA,
      ],
  )
  degp = deg_kernel(dst2)

  blk1 = 2048
  hs, dinv = pl.pallas_call(
      functools.partial(_hs_body, blk1),
      grid=(np_ // blk1,),
      in_specs=[
          pl.BlockSpec((blk1, d), lambda i: (i, 0)),
          pl.BlockSpec((d, d), lambda i: (0, 0)),
          pl.BlockSpec((NC, np_), lambda i: (0, 0)),
      ],
      out_specs=[
          pl.BlockSpec((blk1, d), lambda i: (i, 0)),
          pl.BlockSpec((blk1, 1), lambda i: (i, 0)),
      ],
      out_shape=[
          jax.ShapeDtypeStruct((np_, d), jnp.float32),
          jax.ShapeDtypeStruct((np_, 1), jnp.float32),
      ],
  )(xp, W, degp)

  agg_kernel = pl.kernel(
      functools.partial(_agg_body, k0, b0, r0, b1, r1, rpa, rlast),
      out_type=jax.ShapeDtypeStruct((NC, npa, d), jnp.float32),
      mesh=mesh,
      scratch_types=[
          pltpu.VMEM_SHARED((npa, d), jnp.float32),
          pltpu.VMEM((NI, CHA), jnp.int32),
          pltpu.VMEM((NI, CHA), jnp.int32),
          pltpu.VMEM((NB, CHA, d), jnp.float32),
          pltpu.SemaphoreType.DMA((NI,)),
          pltpu.SemaphoreType.DMA((NI,)),
          pltpu.SemaphoreType.DMA((NB,)),
          pltpu.SemaphoreType.DMA((NB,)),
      ],
  )
  part = agg_kernel(srcr, dstr, hs)

  blk2 = 5000
  out = pl.pallas_call(
      _out_body,
      grid=(n // blk2,),
      in_specs=[
          pl.BlockSpec((NC, blk2, d), lambda i: (0, i, 0)),
          pl.BlockSpec((blk2, d), lambda i: (i, 0)),
          pl.BlockSpec((blk2, 1), lambda i: (i, 0)),
          pl.BlockSpec((1, d), lambda i: (0, 0)),
      ],
      out_specs=pl.BlockSpec((blk2, d), lambda i: (i, 0)),
      out_shape=jax.ShapeDtypeStruct((n, d), jnp.float32),
  )(part, hs, dinv, b.reshape(1, d))

  return out
